# baseline (speedup 1.0000x reference)
import functools

import jax
import jax.numpy as jnp
from jax import lax
from jax.experimental import pallas as pl
from jax.experimental.pallas import tpu as pltpu
from jax.experimental.pallas import tpu_sc as plsc

VOCAB = 100000
EMBED_DIM = 768
BATCH = 4
SEQ_LEN = 2048
NUM_TOKENS = BATCH * SEQ_LEN

_info = plsc.get_sparse_core_info()
NC, NS = _info.num_cores, _info.num_subcores
NW = NC * NS
ROWS_PER_WORKER = NUM_TOKENS // NW
W_PER_BATCH = SEQ_LEN // ROWS_PER_WORKER
CHUNK = 64
N_CHUNKS = ROWS_PER_WORKER // CHUNK
NBUF = 2


def _make_kernel():
    mesh = plsc.VectorSubcoreMesh(core_axis_name="c", subcore_axis_name="s")

    @functools.partial(
        pl.kernel,
        mesh=mesh,
        out_type=jax.ShapeDtypeStruct((BATCH, SEQ_LEN, EMBED_DIM), jnp.float32),
        scratch_types=[
            pltpu.VMEM((ROWS_PER_WORKER,), jnp.int32),
            pltpu.VMEM((NBUF, CHUNK, EMBED_DIM), jnp.float32),
        ]
        + [pltpu.SemaphoreType.DMA] * (2 * NBUF),
    )
    def emb(ids_hbm, table_hbm, out_hbm, idx_v, rows_v, *sems):
        gsems = sems[:NBUF]
        wsems = sems[NBUF:]
        wid = lax.axis_index("s") * NC + lax.axis_index("c")
        b = wid // W_PER_BATCH
        s0 = (wid % W_PER_BATCH) * ROWS_PER_WORKER
        head = ROWS_PER_WORKER // 2
        assert (NBUF - 1) * CHUNK <= head
        pltpu.sync_copy(ids_hbm.at[b, pl.ds(s0, head)],
                        idx_v.at[pl.ds(0, head)])

        def gather(g):
            return pltpu.async_copy(
                table_hbm.at[idx_v.at[pl.ds(g * CHUNK, CHUNK)]],
                rows_v.at[g % NBUF], gsems[g % NBUF])

        gathers = [None] * N_CHUNKS
        writes = [None] * N_CHUNKS
        for g in range(NBUF - 1):
            gathers[g] = gather(g)
        pltpu.sync_copy(ids_hbm.at[b, pl.ds(s0 + head, ROWS_PER_WORKER - head)],
                        idx_v.at[pl.ds(head, ROWS_PER_WORKER - head)])
        for g in range(N_CHUNKS):
            nxt = g + NBUF - 1
            if nxt < N_CHUNKS:
                if writes[nxt - NBUF] is not None:
                    writes[nxt - NBUF].wait()
                gathers[nxt] = gather(nxt)
            gathers[g].wait()
            writes[g] = pltpu.async_copy(
                rows_v.at[g % NBUF],
                out_hbm.at[b, pl.ds(s0 + g * CHUNK, CHUNK)],
                wsems[g % NBUF])
        for g in range(max(0, N_CHUNKS - NBUF), N_CHUNKS):
            writes[g].wait()

    return emb


_emb = _make_kernel()


def kernel(token_ids, table):
    return _emb(token_ids.astype(jnp.int32), table)


# speedup vs baseline: 1.0262x; 1.0262x over previous
import functools

import jax
import jax.numpy as jnp
from jax import lax
from jax.experimental import pallas as pl
from jax.experimental.pallas import tpu as pltpu
from jax.experimental.pallas import tpu_sc as plsc

VOCAB = 100000
EMBED_DIM = 768
BATCH = 4
SEQ_LEN = 2048
NUM_TOKENS = BATCH * SEQ_LEN

_info = plsc.get_sparse_core_info()
NC, NS = _info.num_cores, _info.num_subcores
NW = NC * NS
ROWS_PER_WORKER = NUM_TOKENS // NW
W_PER_BATCH = SEQ_LEN // ROWS_PER_WORKER
CHUNK = 32
N_CHUNKS = ROWS_PER_WORKER // CHUNK
NBUF = 5


def _make_kernel():
    mesh = plsc.VectorSubcoreMesh(core_axis_name="c", subcore_axis_name="s")

    @functools.partial(
        pl.kernel,
        mesh=mesh,
        out_type=jax.ShapeDtypeStruct((BATCH, SEQ_LEN, EMBED_DIM), jnp.float32),
        scratch_types=[
            pltpu.VMEM((ROWS_PER_WORKER,), jnp.int32),
            pltpu.VMEM((NBUF, CHUNK, EMBED_DIM), jnp.float32),
        ]
        + [pltpu.SemaphoreType.DMA] * (2 * NBUF),
    )
    def emb(ids_hbm, table_hbm, out_hbm, idx_v, rows_v, *sems):
        gsems = sems[:NBUF]
        wsems = sems[NBUF:]
        wid = lax.axis_index("s") * NC + lax.axis_index("c")
        b = wid // W_PER_BATCH
        s0 = (wid % W_PER_BATCH) * ROWS_PER_WORKER
        head = ROWS_PER_WORKER // 2
        assert (NBUF - 1) * CHUNK <= head
        pltpu.sync_copy(ids_hbm.at[b, pl.ds(s0, head)],
                        idx_v.at[pl.ds(0, head)])

        def gather(g):
            return pltpu.async_copy(
                table_hbm.at[idx_v.at[pl.ds(g * CHUNK, CHUNK)]],
                rows_v.at[g % NBUF], gsems[g % NBUF])

        gathers = [None] * N_CHUNKS
        writes = [None] * N_CHUNKS
        for g in range(NBUF - 1):
            gathers[g] = gather(g)
        pltpu.sync_copy(ids_hbm.at[b, pl.ds(s0 + head, ROWS_PER_WORKER - head)],
                        idx_v.at[pl.ds(head, ROWS_PER_WORKER - head)])
        for g in range(N_CHUNKS):
            nxt = g + NBUF - 1
            if nxt < N_CHUNKS:
                if writes[nxt - NBUF] is not None:
                    writes[nxt - NBUF].wait()
                gathers[nxt] = gather(nxt)
            gathers[g].wait()
            writes[g] = pltpu.async_copy(
                rows_v.at[g % NBUF],
                out_hbm.at[b, pl.ds(s0 + g * CHUNK, CHUNK)],
                wsems[g % NBUF])
        for g in range(max(0, N_CHUNKS - NBUF), N_CHUNKS):
            writes[g].wait()

    return emb


_emb = _make_kernel()


def kernel(token_ids, table):
    return _emb(token_ids.astype(jnp.int32), table)


# speedup vs baseline: 1.0267x; 1.0005x over previous
import functools

import jax
import jax.numpy as jnp
from jax import lax
from jax.experimental import pallas as pl
from jax.experimental.pallas import tpu as pltpu
from jax.experimental.pallas import tpu_sc as plsc

VOCAB = 100000
EMBED_DIM = 768
BATCH = 4
SEQ_LEN = 2048
NUM_TOKENS = BATCH * SEQ_LEN

_info = plsc.get_sparse_core_info()
NC, NS = _info.num_cores, _info.num_subcores
NW = NC * NS
ROWS_PER_WORKER = NUM_TOKENS // NW
W_PER_BATCH = SEQ_LEN // ROWS_PER_WORKER
CHUNK = 32
N_CHUNKS = ROWS_PER_WORKER // CHUNK
NBUF = 5


def _make_kernel():
    mesh = plsc.VectorSubcoreMesh(core_axis_name="c", subcore_axis_name="s")

    @functools.partial(
        pl.kernel,
        mesh=mesh,
        out_type=jax.ShapeDtypeStruct((BATCH, SEQ_LEN, EMBED_DIM), jnp.float32),
        scratch_types=[
            pltpu.VMEM((ROWS_PER_WORKER,), jnp.int32),
            pltpu.VMEM((NBUF, CHUNK, EMBED_DIM), jnp.float32),
        ]
        + [pltpu.SemaphoreType.DMA] * (2 * NBUF),
    )
    def emb(ids_hbm, table_hbm, out_hbm, idx_v, rows_v, *sems):
        gsems = sems[:NBUF]
        wsems = sems[NBUF:]
        wid = lax.axis_index("s") * NC + lax.axis_index("c")
        b = wid // W_PER_BATCH
        s0 = (wid % W_PER_BATCH) * ROWS_PER_WORKER
        head = ROWS_PER_WORKER // 2
        assert (NBUF - 1) * CHUNK <= head
        pltpu.sync_copy(ids_hbm.at[b, pl.ds(s0, head)],
                        idx_v.at[pl.ds(0, head)])

        def gather(g):
            return pltpu.async_copy(
                table_hbm.at[idx_v.at[pl.ds(g * CHUNK, CHUNK)]],
                rows_v.at[g % NBUF], gsems[g % NBUF], priority=1)

        gathers = [None] * N_CHUNKS
        writes = [None] * N_CHUNKS
        for g in range(NBUF - 1):
            gathers[g] = gather(g)
        pltpu.sync_copy(ids_hbm.at[b, pl.ds(s0 + head, ROWS_PER_WORKER - head)],
                        idx_v.at[pl.ds(head, ROWS_PER_WORKER - head)])
        for g in range(N_CHUNKS):
            nxt = g + NBUF - 1
            if nxt < N_CHUNKS:
                if writes[nxt - NBUF] is not None:
                    writes[nxt - NBUF].wait()
                gathers[nxt] = gather(nxt)
            gathers[g].wait()
            writes[g] = pltpu.async_copy(
                rows_v.at[g % NBUF],
                out_hbm.at[b, pl.ds(s0 + g * CHUNK, CHUNK)],
                wsems[g % NBUF])
        for g in range(max(0, N_CHUNKS - NBUF), N_CHUNKS):
            writes[g].wait()

    return emb


_emb = _make_kernel()


def kernel(token_ids, table):
    return _emb(token_ids.astype(jnp.int32), table)
